# Initial kernel scaffold; baseline (speedup 1.0000x reference)
import jax
import jax.numpy as jnp
from jax.experimental import pallas as pl


def kernel(x, edge_index, edge_attr, xe1, xe2, ee1, ee2, W1, b1, W2, b2, gamma, beta):
    raise NotImplementedError("write your pallas kernel here")


# speedup vs baseline: 3.2474x; 3.2474x over previous
import functools

import jax
import jax.numpy as jnp
from jax import lax
from jax.experimental import pallas as pl
from jax.experimental.pallas import tpu as pltpu
from jax.experimental.pallas import tpu_sc as plsc

N = 10000
E = 160000
EMB = 256
HF = 128
RB = 1000
K = 80
EPT = E // 16

CW = 128
EC = 163840
EPT_C = EC // 32
NC_PAD = 10112
N_PAD = 10240
RPT = N_PAD // 16
RPT_C = NC_PAD // 16


@functools.cache
def _make_segsum():
  mesh = plsc.VectorSubcoreMesh(core_axis_name="c", subcore_axis_name="s")

  @functools.partial(
      pl.kernel,
      out_type=(jax.ShapeDtypeStruct((N_PAD, HF), jnp.float32),
                jax.ShapeDtypeStruct((N_PAD, HF), jnp.float32)),
      mesh=mesh,
      scratch_types=[
          pltpu.VMEM((K,), jnp.int32),
          pltpu.VMEM((K,), jnp.int32),
          pltpu.VMEM((K, HF), jnp.float32),
          pltpu.VMEM_SHARED((N_PAD, HF), jnp.float32),
          pltpu.SemaphoreType.DMA,
      ],
  )
  def segsum(h0_hbm, h1_hbm, src_hbm, dst_hbm, zro_hbm,
             out0_hbm, out1_hbm, idx_s, idx_d, rows, acc, sem):
    cid = lax.axis_index("c")
    sid = lax.axis_index("s")
    pltpu.sync_copy(zro_hbm, acc.at[pl.ds(sid * RPT, RPT)])
    plsc.subcore_barrier()

    def run(h_hbm):
      def step(i, carry):
        base = sid * EPT + i * K
        pltpu.sync_copy(src_hbm.at[pl.ds(base, K)], idx_s)
        pltpu.sync_copy(dst_hbm.at[pl.ds(base, K)], idx_d)
        pltpu.async_copy(h_hbm.at[idx_s], rows, sem).wait()
        pltpu.sync_copy(rows, acc.at[idx_d], add=True)
        return carry
      lax.fori_loop(0, EPT // K, step, 0)

    @pl.when(cid == 0)
    def _():
      run(h0_hbm)

    @pl.when(cid == 1)
    def _():
      run(h1_hbm)

    plsc.subcore_barrier()
    sl = pl.ds(sid * RPT, RPT)

    @pl.when(cid == 0)
    def _():
      pltpu.sync_copy(acc.at[sl], out0_hbm.at[sl])

    @pl.when(cid == 1)
    def _():
      pltpu.sync_copy(acc.at[sl], out1_hbm.at[sl])

  return segsum


@functools.cache
def _make_cbuild():
  mesh = plsc.VectorSubcoreMesh(core_axis_name="c", subcore_axis_name="s")

  @functools.partial(
      pl.kernel,
      out_type=jax.ShapeDtypeStruct((2 * NC_PAD, CW), jnp.float32),
      mesh=mesh,
      scratch_types=[
          pltpu.VMEM((K,), jnp.int32),
          pltpu.VMEM((K,), jnp.int32),
          pltpu.VMEM((K,), jnp.int32),
          pltpu.VMEM((K,), jnp.int32),
          pltpu.VMEM((K, CW), jnp.float32),
          pltpu.VMEM_SHARED((NC_PAD, CW), jnp.float32),
          pltpu.SemaphoreType.DMA,
      ],
  )
  def cbuild(ea0_hbm, ea1_hbm, dstp_hbm, id_hbm, zc_hbm,
             out_hbm, ba, bb, bcat, bd, rows, acc, sem):
    cid = lax.axis_index("c")
    sid = lax.axis_index("s")
    pltpu.sync_copy(zc_hbm, acc.at[pl.ds(sid * RPT_C, RPT_C)])
    plsc.subcore_barrier()
    wid = sid * 2 + cid

    def step(i, carry):
      base = wid * EPT_C + i * K
      pltpu.sync_copy(ea0_hbm.at[pl.ds(base, K)], ba)
      pltpu.sync_copy(ea1_hbm.at[pl.ds(base, K)], bb)
      for j in range(K // 16):
        s = pl.ds(j * 16, 16)
        bcat[s] = ba[s] * 3 + bb[s]
      pltpu.sync_copy(dstp_hbm.at[pl.ds(base, K)], bd)
      pltpu.async_copy(id_hbm.at[bcat], rows, sem).wait()
      pltpu.sync_copy(rows, acc.at[bd], add=True)
      return carry
    lax.fori_loop(0, EPT_C // K, step, 0)

    plsc.subcore_barrier()
    sl = pl.ds(sid * RPT_C, RPT_C)
    pltpu.sync_copy(acc.at[sl],
                    out_hbm.at[pl.ds(cid * NC_PAD + sid * RPT_C, RPT_C)])

  return cbuild


def _segsum(p0, p1, src, dst, zro):
  a0, a1 = _make_segsum()(p0, p1, src, dst, zro)
  return a0[:N], a1[:N]


def _cbuild(ea0p, ea1p, dstp, idtbl, zc):
  cf = _make_cbuild()(ea0p, ea1p, dstp, idtbl, zc)
  return cf[:N], cf[NC_PAD:NC_PAD + N]


def _embed_body(x_ref, t1_ref, t2_ref, o0_ref, o1_ref):
  xb = x_ref[...]
  x0 = xb[:, 0:1]
  x1 = xb[:, 1:2]
  t1 = t1_ref[...]
  t2 = t2_ref[...]
  h = jnp.where(x0 == 0, t1[0:1], jnp.where(x0 == 1, t1[1:2], t1[2:3]))
  h = h + jnp.where(x1 == 0, t2[0:1], jnp.where(x1 == 1, t2[1:2], t2[2:3]))
  o0_ref[...] = h[:, :HF]
  o1_ref[...] = h[:, HF:]


def _embed(x, t1, t2):
  return pl.pallas_call(
      _embed_body,
      grid=(N // RB,),
      in_specs=[pl.BlockSpec((RB, 2), lambda i: (i, 0)),
                pl.BlockSpec((3, EMB), lambda i: (0, 0)),
                pl.BlockSpec((3, EMB), lambda i: (0, 0))],
      out_specs=[pl.BlockSpec((RB, HF), lambda i: (i, 0)),
                 pl.BlockSpec((RB, HF), lambda i: (i, 0))],
      out_shape=[jax.ShapeDtypeStruct((N, HF), jnp.float32)] * 2,
  )(x, t1, t2)


def _mlp_body(a0r, a1r, p0r, p1r, car, cbr, tr, w1r, b1r, w2r, b2r,
              o_ref, s1_ref, s2_ref):
  i = pl.program_id(0)
  cmat = car[...] + cbr[...]
  tv = tr[...]
  ct = jnp.dot(cmat, tv, preferred_element_type=jnp.float32,
               precision=lax.Precision.HIGHEST)
  in0 = a0r[...] + p0r[...] + ct[:, :HF] + tv[12:13, :HF]
  in1 = a1r[...] + p1r[...] + ct[:, HF:] + tv[12:13, HF:]
  agg = jnp.concatenate([in0, in1], axis=1)
  z = jnp.dot(agg, w1r[...], preferred_element_type=jnp.float32) + b1r[...]
  z = jnp.maximum(z, 0.0)
  o = jnp.dot(z, w2r[...], preferred_element_type=jnp.float32) + b2r[...]
  o_ref[...] = o

  @pl.when(i == 0)
  def _():
    s1_ref[...] = jnp.zeros_like(s1_ref)
    s2_ref[...] = jnp.zeros_like(s2_ref)

  s1_ref[...] += jnp.sum(o, axis=0, keepdims=True)
  s2_ref[...] += jnp.sum(o * o, axis=0, keepdims=True)


def _mlp(a0, a1, p0, p1, ca, cb, t, w1, b1, w2, b2):
  return pl.pallas_call(
      _mlp_body,
      grid=(N // RB,),
      in_specs=[pl.BlockSpec((RB, HF), lambda i: (i, 0)),
                pl.BlockSpec((RB, HF), lambda i: (i, 0)),
                pl.BlockSpec((RB, HF), lambda i: (i, 0)),
                pl.BlockSpec((RB, HF), lambda i: (i, 0)),
                pl.BlockSpec((RB, CW), lambda i: (i, 0)),
                pl.BlockSpec((RB, CW), lambda i: (i, 0)),
                pl.BlockSpec((CW, EMB), lambda i: (0, 0)),
                pl.BlockSpec((EMB, 2 * EMB), lambda i: (0, 0)),
                pl.BlockSpec((1, 2 * EMB), lambda i: (0, 0)),
                pl.BlockSpec((2 * EMB, EMB), lambda i: (0, 0)),
                pl.BlockSpec((1, EMB), lambda i: (0, 0))],
      out_specs=[pl.BlockSpec((RB, EMB), lambda i: (i, 0)),
                 pl.BlockSpec((1, EMB), lambda i: (0, 0)),
                 pl.BlockSpec((1, EMB), lambda i: (0, 0))],
      out_shape=[jax.ShapeDtypeStruct((N, EMB), jnp.float32),
                 jax.ShapeDtypeStruct((1, EMB), jnp.float32),
                 jax.ShapeDtypeStruct((1, EMB), jnp.float32)],
  )(a0, a1, p0, p1, ca, cb, t, w1, b1, w2, b2)


def _norm_mid_body(orf, s1r, s2r, gr, br, o0_ref, o1_ref):
  m = s1r[...] * (1.0 / N)
  v = s2r[...] * (1.0 / N) - m * m
  scale = lax.rsqrt(v + 1e-5) * gr[...]
  y = (orf[...] - m) * scale + br[...]
  y = jnp.maximum(y, 0.0)
  o0_ref[...] = y[:, :HF]
  o1_ref[...] = y[:, HF:]


def _norm_last_body(orf, s1r, s2r, gr, br, o_ref):
  m = s1r[...] * (1.0 / N)
  v = s2r[...] * (1.0 / N) - m * m
  scale = lax.rsqrt(v + 1e-5) * gr[...]
  o_ref[...] = (orf[...] - m) * scale + br[...]


def _norm(o, s1, s2, g, b, last):
  in_specs = [pl.BlockSpec((RB, EMB), lambda i: (i, 0)),
              pl.BlockSpec((1, EMB), lambda i: (0, 0)),
              pl.BlockSpec((1, EMB), lambda i: (0, 0)),
              pl.BlockSpec((1, EMB), lambda i: (0, 0)),
              pl.BlockSpec((1, EMB), lambda i: (0, 0))]
  if last:
    return pl.pallas_call(
        _norm_last_body,
        grid=(N // RB,),
        in_specs=in_specs,
        out_specs=[pl.BlockSpec((RB, EMB), lambda i: (i, 0))],
        out_shape=[jax.ShapeDtypeStruct((N, EMB), jnp.float32)],
    )(o, s1, s2, g, b)[0]
  return pl.pallas_call(
      _norm_mid_body,
      grid=(N // RB,),
      in_specs=in_specs,
      out_specs=[pl.BlockSpec((RB, HF), lambda i: (i, 0)),
                 pl.BlockSpec((RB, HF), lambda i: (i, 0))],
      out_shape=[jax.ShapeDtypeStruct((N, HF), jnp.float32)] * 2,
  )(o, s1, s2, g, b)


def kernel(x, edge_index, edge_attr, xe1, xe2, ee1, ee2, W1, b1, W2, b2,
           gamma, beta):
  x = x.astype(jnp.int32)
  src = edge_index[0].astype(jnp.int32)
  dst = edge_index[1].astype(jnp.int32)
  ea0 = edge_attr[:, 0].astype(jnp.int32)
  ea1 = edge_attr[:, 1].astype(jnp.int32)

  npad = EC - E
  ea0p = jnp.concatenate([ea0, jnp.zeros((npad,), jnp.int32)])
  ea1p = jnp.concatenate([ea1, jnp.zeros((npad,), jnp.int32)])
  dstp = jnp.concatenate(
      [dst, N + (jnp.arange(npad, dtype=jnp.int32) % 112)])

  idtbl = jnp.eye(CW, dtype=jnp.float32)
  zero_seg = jnp.zeros((RPT, HF), jnp.float32)
  zero_cnt = jnp.zeros((RPT_C, CW), jnp.float32)

  tfull = (ee1[:, :, None, :] + ee2[:, None, :, :]).reshape(3, 18, EMB)
  tpad = jnp.concatenate(
      [tfull, jnp.zeros((3, CW - 18, EMB), jnp.float32)], axis=1)

  p0, p1 = _embed(x, xe1[:3], xe2)
  ca, cb = _cbuild(ea0p, ea1p, dstp, idtbl, zero_cnt)

  out = None
  for l in range(3):
    a0, a1 = _segsum(p0, p1, src, dst, zero_seg)
    o, s1, s2 = _mlp(a0, a1, p0, p1, ca, cb, tpad[l],
                     W1[l], b1[l][None], W2[l], b2[l][None])
    if l < 2:
      p0, p1 = _norm(o, s1, s2, gamma[l][None], beta[l][None], False)
    else:
      out = _norm(o, s1, s2, gamma[l][None], beta[l][None], True)
  return out


# speedup vs baseline: 5.0969x; 1.5695x over previous
import functools

import jax
import jax.numpy as jnp
from jax import lax
from jax.experimental import pallas as pl
from jax.experimental.pallas import tpu as pltpu
from jax.experimental.pallas import tpu_sc as plsc

N = 10000
E = 160000
EMB = 256
HF = 128
RB = 1000
K = 80
EPT = E // 16

CW = 128
EC = 163840
EPT_C = EC // 32
NC_PAD = 10112
N_PAD = 10240
RPT = N_PAD // 16
RPT_C = NC_PAD // 16


@functools.cache
def _make_segsum():
  mesh = plsc.VectorSubcoreMesh(core_axis_name="c", subcore_axis_name="s")

  @functools.partial(
      pl.kernel,
      out_type=(jax.ShapeDtypeStruct((N_PAD, HF), jnp.float32),
                jax.ShapeDtypeStruct((N_PAD, HF), jnp.float32)),
      mesh=mesh,
      scratch_types=[
          pltpu.VMEM((K,), jnp.int32),
          pltpu.VMEM((K,), jnp.int32),
          pltpu.VMEM((K, HF), jnp.float32),
          pltpu.VMEM_SHARED((N_PAD, HF), jnp.float32),
          pltpu.SemaphoreType.DMA,
      ],
  )
  def segsum(h0_hbm, h1_hbm, src_hbm, dst_hbm, zro_hbm,
             out0_hbm, out1_hbm, idx_s, idx_d, rows, acc, sem):
    cid = lax.axis_index("c")
    sid = lax.axis_index("s")
    pltpu.sync_copy(zro_hbm, acc.at[pl.ds(sid * RPT, RPT)])
    plsc.subcore_barrier()

    def run(h_hbm):
      def step(i, carry):
        base = sid * EPT + i * K
        pltpu.sync_copy(src_hbm.at[pl.ds(base, K)], idx_s)
        pltpu.sync_copy(dst_hbm.at[pl.ds(base, K)], idx_d)
        pltpu.async_copy(h_hbm.at[idx_s], rows, sem).wait()
        pltpu.sync_copy(rows, acc.at[idx_d], add=True)
        return carry
      lax.fori_loop(0, EPT // K, step, 0)

    @pl.when(cid == 0)
    def _():
      run(h0_hbm)

    @pl.when(cid == 1)
    def _():
      run(h1_hbm)

    plsc.subcore_barrier()
    sl = pl.ds(sid * RPT, RPT)

    @pl.when(cid == 0)
    def _():
      pltpu.sync_copy(acc.at[sl], out0_hbm.at[sl])

    @pl.when(cid == 1)
    def _():
      pltpu.sync_copy(acc.at[sl], out1_hbm.at[sl])

  return segsum


@functools.cache
def _make_cbuild():
  mesh = plsc.VectorSubcoreMesh(core_axis_name="c", subcore_axis_name="s")

  @functools.partial(
      pl.kernel,
      out_type=jax.ShapeDtypeStruct((2 * NC_PAD, CW), jnp.float32),
      mesh=mesh,
      scratch_types=[
          pltpu.VMEM((K,), jnp.int32),
          pltpu.VMEM((K,), jnp.int32),
          pltpu.VMEM((K,), jnp.int32),
          pltpu.VMEM((K,), jnp.int32),
          pltpu.VMEM((K,), jnp.int32),
          pltpu.VMEM((K, CW), jnp.float32),
          pltpu.VMEM_SHARED((NC_PAD, CW), jnp.float32),
          pltpu.SemaphoreType.DMA,
      ],
  )
  def cbuild(ea0_hbm, ea1_hbm, dstp_hbm, id_hbm, off_hbm, zc_hbm,
             out_hbm, ba, bb, bcat, bd, boff, rows, acc, sem):
    cid = lax.axis_index("c")
    sid = lax.axis_index("s")
    pltpu.sync_copy(zc_hbm, acc.at[pl.ds(sid * RPT_C, RPT_C)])
    pltpu.sync_copy(off_hbm, boff)
    plsc.subcore_barrier()
    wid = sid * 2 + cid

    def step(i, carry):
      base = wid * EPT_C + i * K
      pltpu.sync_copy(ea0_hbm.at[pl.ds(base, K)], ba)
      pltpu.sync_copy(ea1_hbm.at[pl.ds(base, K)], bb)
      for j in range(K // 16):
        s = pl.ds(j * 16, 16)
        bcat[s] = ba[s] * 3 + bb[s] + boff[s]
      pltpu.sync_copy(dstp_hbm.at[pl.ds(base, K)], bd)
      pltpu.async_copy(id_hbm.at[bcat], rows, sem).wait()
      pltpu.sync_copy(rows, acc.at[bd], add=True)
      return carry
    lax.fori_loop(0, EPT_C // K, step, 0)

    plsc.subcore_barrier()
    sl = pl.ds(sid * RPT_C, RPT_C)
    pltpu.sync_copy(acc.at[sl],
                    out_hbm.at[pl.ds(cid * NC_PAD + sid * RPT_C, RPT_C)])

  return cbuild


def _segsum(p0, p1, src, dst, zro):
  a0, a1 = _make_segsum()(p0, p1, src, dst, zro)
  return a0[:N], a1[:N]


def _cbuild(ea0p, ea1p, dstp, idtbl, off, zc):
  cf = _make_cbuild()(ea0p, ea1p, dstp, idtbl, off, zc)
  return cf[:N], cf[NC_PAD:NC_PAD + N]


def _embed_body(x_ref, t1_ref, t2_ref, o0_ref, o1_ref):
  xb = x_ref[...]
  x0 = xb[:, 0:1]
  x1 = xb[:, 1:2]
  t1 = t1_ref[...]
  t2 = t2_ref[...]
  h = jnp.where(x0 == 0, t1[0:1], jnp.where(x0 == 1, t1[1:2], t1[2:3]))
  h = h + jnp.where(x1 == 0, t2[0:1], jnp.where(x1 == 1, t2[1:2], t2[2:3]))
  o0_ref[...] = h[:, :HF]
  o1_ref[...] = h[:, HF:]


def _embed(x, t1, t2):
  return pl.pallas_call(
      _embed_body,
      grid=(N // RB,),
      in_specs=[pl.BlockSpec((RB, 2), lambda i: (i, 0)),
                pl.BlockSpec((3, EMB), lambda i: (0, 0)),
                pl.BlockSpec((3, EMB), lambda i: (0, 0))],
      out_specs=[pl.BlockSpec((RB, HF), lambda i: (i, 0)),
                 pl.BlockSpec((RB, HF), lambda i: (i, 0))],
      out_shape=[jax.ShapeDtypeStruct((N, HF), jnp.float32)] * 2,
  )(x, t1, t2)


def _mlp_body(a0r, a1r, p0r, p1r, car, cbr, tr, w1r, b1r, w2r, b2r,
              o_ref, s1_ref, s2_ref):
  i = pl.program_id(0)
  cmat = car[...] + cbr[...]
  tv = tr[...]
  ct = jnp.dot(cmat, tv, preferred_element_type=jnp.float32,
               precision=lax.Precision.HIGHEST)
  in0 = a0r[...] + p0r[...] + ct[:, :HF] + tv[12:13, :HF]
  in1 = a1r[...] + p1r[...] + ct[:, HF:] + tv[12:13, HF:]
  agg = jnp.concatenate([in0, in1], axis=1)
  z = jnp.dot(agg, w1r[...], preferred_element_type=jnp.float32) + b1r[...]
  z = jnp.maximum(z, 0.0)
  o = jnp.dot(z, w2r[...], preferred_element_type=jnp.float32) + b2r[...]
  o_ref[...] = o

  @pl.when(i == 0)
  def _():
    s1_ref[...] = jnp.zeros_like(s1_ref)
    s2_ref[...] = jnp.zeros_like(s2_ref)

  s1_ref[...] += jnp.sum(o, axis=0, keepdims=True)
  s2_ref[...] += jnp.sum(o * o, axis=0, keepdims=True)


def _mlp(a0, a1, p0, p1, ca, cb, t, w1, b1, w2, b2):
  return pl.pallas_call(
      _mlp_body,
      grid=(N // RB,),
      in_specs=[pl.BlockSpec((RB, HF), lambda i: (i, 0)),
                pl.BlockSpec((RB, HF), lambda i: (i, 0)),
                pl.BlockSpec((RB, HF), lambda i: (i, 0)),
                pl.BlockSpec((RB, HF), lambda i: (i, 0)),
                pl.BlockSpec((RB, CW), lambda i: (i, 0)),
                pl.BlockSpec((RB, CW), lambda i: (i, 0)),
                pl.BlockSpec((CW, EMB), lambda i: (0, 0)),
                pl.BlockSpec((EMB, 2 * EMB), lambda i: (0, 0)),
                pl.BlockSpec((1, 2 * EMB), lambda i: (0, 0)),
                pl.BlockSpec((2 * EMB, EMB), lambda i: (0, 0)),
                pl.BlockSpec((1, EMB), lambda i: (0, 0))],
      out_specs=[pl.BlockSpec((RB, EMB), lambda i: (i, 0)),
                 pl.BlockSpec((1, EMB), lambda i: (0, 0)),
                 pl.BlockSpec((1, EMB), lambda i: (0, 0))],
      out_shape=[jax.ShapeDtypeStruct((N, EMB), jnp.float32),
                 jax.ShapeDtypeStruct((1, EMB), jnp.float32),
                 jax.ShapeDtypeStruct((1, EMB), jnp.float32)],
  )(a0, a1, p0, p1, ca, cb, t, w1, b1, w2, b2)


def _norm_mid_body(orf, s1r, s2r, gr, br, o0_ref, o1_ref):
  m = s1r[...] * (1.0 / N)
  v = s2r[...] * (1.0 / N) - m * m
  scale = lax.rsqrt(v + 1e-5) * gr[...]
  y = (orf[...] - m) * scale + br[...]
  y = jnp.maximum(y, 0.0)
  o0_ref[...] = y[:, :HF]
  o1_ref[...] = y[:, HF:]


def _norm_last_body(orf, s1r, s2r, gr, br, o_ref):
  m = s1r[...] * (1.0 / N)
  v = s2r[...] * (1.0 / N) - m * m
  scale = lax.rsqrt(v + 1e-5) * gr[...]
  o_ref[...] = (orf[...] - m) * scale + br[...]


def _norm(o, s1, s2, g, b, last):
  in_specs = [pl.BlockSpec((RB, EMB), lambda i: (i, 0)),
              pl.BlockSpec((1, EMB), lambda i: (0, 0)),
              pl.BlockSpec((1, EMB), lambda i: (0, 0)),
              pl.BlockSpec((1, EMB), lambda i: (0, 0)),
              pl.BlockSpec((1, EMB), lambda i: (0, 0))]
  if last:
    return pl.pallas_call(
        _norm_last_body,
        grid=(N // RB,),
        in_specs=in_specs,
        out_specs=[pl.BlockSpec((RB, EMB), lambda i: (i, 0))],
        out_shape=[jax.ShapeDtypeStruct((N, EMB), jnp.float32)],
    )(o, s1, s2, g, b)[0]
  return pl.pallas_call(
      _norm_mid_body,
      grid=(N // RB,),
      in_specs=in_specs,
      out_specs=[pl.BlockSpec((RB, HF), lambda i: (i, 0)),
                 pl.BlockSpec((RB, HF), lambda i: (i, 0))],
      out_shape=[jax.ShapeDtypeStruct((N, HF), jnp.float32)] * 2,
  )(o, s1, s2, g, b)


def kernel(x, edge_index, edge_attr, xe1, xe2, ee1, ee2, W1, b1, W2, b2,
           gamma, beta):
  x = x.astype(jnp.int32)
  src = edge_index[0].astype(jnp.int32)
  dst = edge_index[1].astype(jnp.int32)
  ea0 = edge_attr[:, 0].astype(jnp.int32)
  ea1 = edge_attr[:, 1].astype(jnp.int32)

  npad = EC - E
  ea0p = jnp.concatenate([ea0, jnp.zeros((npad,), jnp.int32)])
  ea1p = jnp.concatenate([ea1, jnp.zeros((npad,), jnp.int32)])
  dstp = jnp.concatenate(
      [dst, N + (jnp.arange(npad, dtype=jnp.int32) % 112)])

  idtbl = jnp.tile(jnp.eye(18, CW, dtype=jnp.float32), (K, 1))
  katoff = 18 * jnp.arange(K, dtype=jnp.int32)
  zero_seg = jnp.zeros((RPT, HF), jnp.float32)
  zero_cnt = jnp.zeros((RPT_C, CW), jnp.float32)

  tfull = (ee1[:, :, None, :] + ee2[:, None, :, :]).reshape(3, 18, EMB)
  tpad = jnp.concatenate(
      [tfull, jnp.zeros((3, CW - 18, EMB), jnp.float32)], axis=1)

  p0, p1 = _embed(x, xe1[:3], xe2)
  ca, cb = _cbuild(ea0p, ea1p, dstp, idtbl, katoff, zero_cnt)

  out = None
  for l in range(3):
    a0, a1 = _segsum(p0, p1, src, dst, zero_seg)
    o, s1, s2 = _mlp(a0, a1, p0, p1, ca, cb, tpad[l],
                     W1[l], b1[l][None], W2[l], b2[l][None])
    if l < 2:
      p0, p1 = _norm(o, s1, s2, gamma[l][None], beta[l][None], False)
    else:
      out = _norm(o, s1, s2, gamma[l][None], beta[l][None], True)
  return out


# speedup vs baseline: 8.0950x; 1.5882x over previous
import functools

import jax
import jax.numpy as jnp
from jax import lax
from jax.experimental import pallas as pl
from jax.experimental.pallas import tpu as pltpu
from jax.experimental.pallas import tpu_sc as plsc

N = 10000
E = 160000
EMB = 256
HF = 128
RB = 1000
K = 80
EPT = E // 16

CW = 128
EC = 163840
EPT_C = EC // 32
NC_PAD = 10112
N_PAD = 10240
RPT = N_PAD // 16
RPT_C = NC_PAD // 16


NCH = EPT // K


@functools.cache
def _make_segsum():
  mesh = plsc.VectorSubcoreMesh(core_axis_name="c", subcore_axis_name="s")

  @functools.partial(
      pl.kernel,
      out_type=(jax.ShapeDtypeStruct((N_PAD, HF), jnp.float32),
                jax.ShapeDtypeStruct((N_PAD, HF), jnp.float32)),
      mesh=mesh,
      scratch_types=[
          pltpu.VMEM((K,), jnp.int32),
          pltpu.VMEM((K,), jnp.int32),
          pltpu.VMEM((K,), jnp.int32),
          pltpu.VMEM((K,), jnp.int32),
          pltpu.VMEM((K, HF), jnp.float32),
          pltpu.VMEM((K, HF), jnp.float32),
          pltpu.VMEM_SHARED((N_PAD, HF), jnp.float32),
          pltpu.SemaphoreType.DMA,
          pltpu.SemaphoreType.DMA,
          pltpu.SemaphoreType.DMA,
          pltpu.SemaphoreType.DMA,
      ],
  )
  def segsum(h0_hbm, h1_hbm, src_hbm, dst_hbm, zro_hbm,
             out0_hbm, out1_hbm, isa, ida, isb, idb, rows_a, rows_b, acc,
             sem_a, sem_b, sem_ia, sem_ib):
    cid = lax.axis_index("c")
    sid = lax.axis_index("s")
    pltpu.sync_copy(zro_hbm, acc.at[pl.ds(sid * RPT, RPT)])
    plsc.subcore_barrier()

    def run(h_hbm):
      base0 = sid * EPT
      pltpu.sync_copy(src_hbm.at[pl.ds(base0, K)], isa)
      pltpu.sync_copy(dst_hbm.at[pl.ds(base0, K)], ida)
      pltpu.sync_copy(src_hbm.at[pl.ds(base0 + K, K)], isb)
      pltpu.sync_copy(dst_hbm.at[pl.ds(base0 + K, K)], idb)
      pltpu.async_copy(h_hbm.at[isa], rows_a, sem_a)
      pltpu.async_copy(h_hbm.at[isb], rows_b, sem_b)

      def step2(j, carry):
        a = 2 * j
        b = a + 1
        pltpu.make_async_copy(h_hbm.at[isa], rows_a, sem_a).wait()
        pltpu.sync_copy(rows_a, acc.at[ida], add=True)
        base_a = sid * EPT + (a + 2) * K
        pltpu.async_copy(src_hbm.at[pl.ds(base_a, K)], isa, sem_ia)
        pltpu.async_copy(dst_hbm.at[pl.ds(base_a, K)], ida, sem_ia)
        pltpu.make_async_copy(src_hbm.at[pl.ds(base_a, K)], isa, sem_ia).wait()
        pltpu.make_async_copy(dst_hbm.at[pl.ds(base_a, K)], ida, sem_ia).wait()
        pltpu.async_copy(h_hbm.at[isa], rows_a, sem_a)
        pltpu.make_async_copy(h_hbm.at[isb], rows_b, sem_b).wait()
        pltpu.sync_copy(rows_b, acc.at[idb], add=True)

        @pl.when(b + 2 < NCH)
        def _():
          base_b = sid * EPT + (b + 2) * K
          pltpu.async_copy(src_hbm.at[pl.ds(base_b, K)], isb, sem_ib)
          pltpu.async_copy(dst_hbm.at[pl.ds(base_b, K)], idb, sem_ib)
          pltpu.make_async_copy(src_hbm.at[pl.ds(base_b, K)], isb, sem_ib).wait()
          pltpu.make_async_copy(dst_hbm.at[pl.ds(base_b, K)], idb, sem_ib).wait()
          pltpu.async_copy(h_hbm.at[isb], rows_b, sem_b)
        return carry
      lax.fori_loop(0, (NCH - 1) // 2, step2, 0)
      pltpu.make_async_copy(h_hbm.at[isa], rows_a, sem_a).wait()
      pltpu.sync_copy(rows_a, acc.at[ida], add=True)

    @pl.when(cid == 0)
    def _():
      run(h0_hbm)

    @pl.when(cid == 1)
    def _():
      run(h1_hbm)

    plsc.subcore_barrier()
    sl = pl.ds(sid * RPT, RPT)

    @pl.when(cid == 0)
    def _():
      pltpu.sync_copy(acc.at[sl], out0_hbm.at[sl])

    @pl.when(cid == 1)
    def _():
      pltpu.sync_copy(acc.at[sl], out1_hbm.at[sl])

  return segsum


@functools.cache
def _make_cbuild():
  mesh = plsc.VectorSubcoreMesh(core_axis_name="c", subcore_axis_name="s")

  @functools.partial(
      pl.kernel,
      out_type=jax.ShapeDtypeStruct((2 * NC_PAD, CW), jnp.float32),
      mesh=mesh,
      scratch_types=[
          pltpu.VMEM((K,), jnp.int32),
          pltpu.VMEM((K,), jnp.int32),
          pltpu.VMEM((K,), jnp.int32),
          pltpu.VMEM((K,), jnp.int32),
          pltpu.VMEM((K,), jnp.int32),
          pltpu.VMEM((K, CW), jnp.float32),
          pltpu.VMEM_SHARED((NC_PAD, CW), jnp.float32),
          pltpu.SemaphoreType.DMA,
      ],
  )
  def cbuild(ea0_hbm, ea1_hbm, dstp_hbm, id_hbm, off_hbm, zc_hbm,
             out_hbm, ba, bb, bcat, bd, boff, rows, acc, sem):
    cid = lax.axis_index("c")
    sid = lax.axis_index("s")
    pltpu.sync_copy(zc_hbm, acc.at[pl.ds(sid * RPT_C, RPT_C)])
    pltpu.sync_copy(off_hbm, boff)
    plsc.subcore_barrier()
    wid = sid * 2 + cid

    def step(i, carry):
      base = wid * EPT_C + i * K
      pltpu.sync_copy(ea0_hbm.at[pl.ds(base, K)], ba)
      pltpu.sync_copy(ea1_hbm.at[pl.ds(base, K)], bb)
      for j in range(K // 16):
        s = pl.ds(j * 16, 16)
        bcat[s] = ba[s] * 3 + bb[s] + boff[s]
      pltpu.sync_copy(dstp_hbm.at[pl.ds(base, K)], bd)
      pltpu.async_copy(id_hbm.at[bcat], rows, sem).wait()
      pltpu.sync_copy(rows, acc.at[bd], add=True)
      return carry
    lax.fori_loop(0, EPT_C // K, step, 0)

    plsc.subcore_barrier()
    sl = pl.ds(sid * RPT_C, RPT_C)
    pltpu.sync_copy(acc.at[sl],
                    out_hbm.at[pl.ds(cid * NC_PAD + sid * RPT_C, RPT_C)])

  return cbuild


def _segsum(p0, p1, src, dst, zro):
  a0, a1 = _make_segsum()(p0, p1, src, dst, zro)
  return a0[:N], a1[:N]


def _cbuild(ea0p, ea1p, dstp, idtbl, off, zc):
  cf = _make_cbuild()(ea0p, ea1p, dstp, idtbl, off, zc)
  return cf[:N], cf[NC_PAD:NC_PAD + N]


def _embed_body(x_ref, t1_ref, t2_ref, o0_ref, o1_ref):
  xb = x_ref[...]
  x0 = xb[:, 0:1]
  x1 = xb[:, 1:2]
  t1 = t1_ref[...]
  t2 = t2_ref[...]
  h = jnp.where(x0 == 0, t1[0:1], jnp.where(x0 == 1, t1[1:2], t1[2:3]))
  h = h + jnp.where(x1 == 0, t2[0:1], jnp.where(x1 == 1, t2[1:2], t2[2:3]))
  o0_ref[...] = h[:, :HF]
  o1_ref[...] = h[:, HF:]


def _embed(x, t1, t2):
  return pl.pallas_call(
      _embed_body,
      grid=(N // RB,),
      in_specs=[pl.BlockSpec((RB, 2), lambda i: (i, 0)),
                pl.BlockSpec((3, EMB), lambda i: (0, 0)),
                pl.BlockSpec((3, EMB), lambda i: (0, 0))],
      out_specs=[pl.BlockSpec((RB, HF), lambda i: (i, 0)),
                 pl.BlockSpec((RB, HF), lambda i: (i, 0))],
      out_shape=[jax.ShapeDtypeStruct((N, HF), jnp.float32)] * 2,
  )(x, t1, t2)


def _mlp_body(a0r, a1r, p0r, p1r, car, cbr, tr, w1r, b1r, w2r, b2r,
              o_ref, s1_ref, s2_ref):
  i = pl.program_id(0)
  cmat = car[...] + cbr[...]
  tv = tr[...]
  ct = jnp.dot(cmat, tv, preferred_element_type=jnp.float32,
               precision=lax.Precision.HIGHEST)
  in0 = a0r[...] + p0r[...] + ct[:, :HF] + tv[12:13, :HF]
  in1 = a1r[...] + p1r[...] + ct[:, HF:] + tv[12:13, HF:]
  agg = jnp.concatenate([in0, in1], axis=1)
  z = jnp.dot(agg, w1r[...], preferred_element_type=jnp.float32) + b1r[...]
  z = jnp.maximum(z, 0.0)
  o = jnp.dot(z, w2r[...], preferred_element_type=jnp.float32) + b2r[...]
  o_ref[...] = o

  @pl.when(i == 0)
  def _():
    s1_ref[...] = jnp.zeros_like(s1_ref)
    s2_ref[...] = jnp.zeros_like(s2_ref)

  s1_ref[...] += jnp.sum(o, axis=0, keepdims=True)
  s2_ref[...] += jnp.sum(o * o, axis=0, keepdims=True)


def _mlp(a0, a1, p0, p1, ca, cb, t, w1, b1, w2, b2):
  return pl.pallas_call(
      _mlp_body,
      grid=(N // RB,),
      in_specs=[pl.BlockSpec((RB, HF), lambda i: (i, 0)),
                pl.BlockSpec((RB, HF), lambda i: (i, 0)),
                pl.BlockSpec((RB, HF), lambda i: (i, 0)),
                pl.BlockSpec((RB, HF), lambda i: (i, 0)),
                pl.BlockSpec((RB, CW), lambda i: (i, 0)),
                pl.BlockSpec((RB, CW), lambda i: (i, 0)),
                pl.BlockSpec((CW, EMB), lambda i: (0, 0)),
                pl.BlockSpec((EMB, 2 * EMB), lambda i: (0, 0)),
                pl.BlockSpec((1, 2 * EMB), lambda i: (0, 0)),
                pl.BlockSpec((2 * EMB, EMB), lambda i: (0, 0)),
                pl.BlockSpec((1, EMB), lambda i: (0, 0))],
      out_specs=[pl.BlockSpec((RB, EMB), lambda i: (i, 0)),
                 pl.BlockSpec((1, EMB), lambda i: (0, 0)),
                 pl.BlockSpec((1, EMB), lambda i: (0, 0))],
      out_shape=[jax.ShapeDtypeStruct((N, EMB), jnp.float32),
                 jax.ShapeDtypeStruct((1, EMB), jnp.float32),
                 jax.ShapeDtypeStruct((1, EMB), jnp.float32)],
  )(a0, a1, p0, p1, ca, cb, t, w1, b1, w2, b2)


def _norm_mid_body(orf, s1r, s2r, gr, br, o0_ref, o1_ref):
  m = s1r[...] * (1.0 / N)
  v = s2r[...] * (1.0 / N) - m * m
  scale = lax.rsqrt(v + 1e-5) * gr[...]
  y = (orf[...] - m) * scale + br[...]
  y = jnp.maximum(y, 0.0)
  o0_ref[...] = y[:, :HF]
  o1_ref[...] = y[:, HF:]


def _norm_last_body(orf, s1r, s2r, gr, br, o_ref):
  m = s1r[...] * (1.0 / N)
  v = s2r[...] * (1.0 / N) - m * m
  scale = lax.rsqrt(v + 1e-5) * gr[...]
  o_ref[...] = (orf[...] - m) * scale + br[...]


def _norm(o, s1, s2, g, b, last):
  in_specs = [pl.BlockSpec((RB, EMB), lambda i: (i, 0)),
              pl.BlockSpec((1, EMB), lambda i: (0, 0)),
              pl.BlockSpec((1, EMB), lambda i: (0, 0)),
              pl.BlockSpec((1, EMB), lambda i: (0, 0)),
              pl.BlockSpec((1, EMB), lambda i: (0, 0))]
  if last:
    return pl.pallas_call(
        _norm_last_body,
        grid=(N // RB,),
        in_specs=in_specs,
        out_specs=[pl.BlockSpec((RB, EMB), lambda i: (i, 0))],
        out_shape=[jax.ShapeDtypeStruct((N, EMB), jnp.float32)],
    )(o, s1, s2, g, b)[0]
  return pl.pallas_call(
      _norm_mid_body,
      grid=(N // RB,),
      in_specs=in_specs,
      out_specs=[pl.BlockSpec((RB, HF), lambda i: (i, 0)),
                 pl.BlockSpec((RB, HF), lambda i: (i, 0))],
      out_shape=[jax.ShapeDtypeStruct((N, HF), jnp.float32)] * 2,
  )(o, s1, s2, g, b)


def kernel(x, edge_index, edge_attr, xe1, xe2, ee1, ee2, W1, b1, W2, b2,
           gamma, beta):
  x = x.astype(jnp.int32)
  src = edge_index[0].astype(jnp.int32)
  dst = edge_index[1].astype(jnp.int32)
  ea0 = edge_attr[:, 0].astype(jnp.int32)
  ea1 = edge_attr[:, 1].astype(jnp.int32)

  npad = EC - E
  ea0p = jnp.concatenate([ea0, jnp.zeros((npad,), jnp.int32)])
  ea1p = jnp.concatenate([ea1, jnp.zeros((npad,), jnp.int32)])
  dstp = jnp.concatenate(
      [dst, N + (jnp.arange(npad, dtype=jnp.int32) % 112)])

  idtbl = jnp.tile(jnp.eye(18, CW, dtype=jnp.float32), (K, 1))
  katoff = 18 * jnp.arange(K, dtype=jnp.int32)
  zero_seg = jnp.zeros((RPT, HF), jnp.float32)
  zero_cnt = jnp.zeros((RPT_C, CW), jnp.float32)

  tfull = (ee1[:, :, None, :] + ee2[:, None, :, :]).reshape(3, 18, EMB)
  tpad = jnp.concatenate(
      [tfull, jnp.zeros((3, CW - 18, EMB), jnp.float32)], axis=1)

  p0, p1 = _embed(x, xe1[:3], xe2)
  ca, cb = _cbuild(ea0p, ea1p, dstp, idtbl, katoff, zero_cnt)

  out = None
  for l in range(3):
    a0, a1 = _segsum(p0, p1, src, dst, zero_seg)
    o, s1, s2 = _mlp(a0, a1, p0, p1, ca, cb, tpad[l],
                     W1[l], b1[l][None], W2[l], b2[l][None])
    if l < 2:
      p0, p1 = _norm(o, s1, s2, gamma[l][None], beta[l][None], False)
    else:
      out = _norm(o, s1, s2, gamma[l][None], beta[l][None], True)
  return out
